# baseline (speedup 1.0000x reference)
import jax
import jax.numpy as jnp
from jax.experimental import pallas as pl
from jax.experimental.pallas import tpu as pltpu

_NUM_CHUNKS = 8


def _dma_copy(w_ref, o_ref, sem):
    seq_len = o_ref.shape[0]
    chunk = seq_len // _NUM_CHUNKS
    copies = [
        pltpu.make_async_copy(
            w_ref.at[pl.ds(i * chunk, chunk), :],
            o_ref.at[pl.ds(i * chunk, chunk), :],
            sem.at[i],
        )
        for i in range(_NUM_CHUNKS)
    ]
    for c in copies:
        c.start()
    for c in copies:
        c.wait()


def kernel(x, weight):
    seq_len = x.shape[1]
    hidden = weight.shape[1]
    out = pl.pallas_call(
        _dma_copy,
        in_specs=[pl.BlockSpec(memory_space=pl.ANY)],
        out_specs=pl.BlockSpec(memory_space=pl.ANY),
        out_shape=jax.ShapeDtypeStruct((seq_len, hidden), weight.dtype),
        scratch_shapes=[pltpu.SemaphoreType.DMA((_NUM_CHUNKS,))],
    )(weight)
    return out[None, :, :]


# speedup vs baseline: 48.8110x; 48.8110x over previous
import jax
import jax.numpy as jnp
from jax.experimental import pallas as pl

_BLOCK_ROWS = 1024


def _copy_block(w_ref, o_ref):
    o_ref[...] = w_ref[...]


def kernel(x, weight):
    seq_len = x.shape[1]
    hidden = weight.shape[1]
    grid = (seq_len // _BLOCK_ROWS,)
    out = pl.pallas_call(
        _copy_block,
        grid=grid,
        in_specs=[pl.BlockSpec((_BLOCK_ROWS, hidden), lambda i: (i, 0))],
        out_specs=pl.BlockSpec((_BLOCK_ROWS, hidden), lambda i: (i, 0)),
        out_shape=jax.ShapeDtypeStruct((seq_len, hidden), weight.dtype),
    )(weight)
    return out[None, :, :]
